# baseline (speedup 1.0000x reference)
import jax
import jax.numpy as jnp
from jax.experimental import pallas as pl
from jax.experimental.pallas import tpu as pltpu

_K = 1024
_D = 64
_BETA = 0.25
_TB = 2048


def _vq_block(lat_ref, w_ref, q_ref, idx_ref, acc_ref, w2_ref, wbf_ref,
              iotaf_ref):
    b = pl.program_id(0)
    t = pl.program_id(1)

    @pl.when(jnp.logical_and(b == 0, t == 0))
    def _():
        w0 = w_ref[...]
        w2_ref[...] = jnp.sum(w0 * w0, axis=1)[None, :]
        wbf_ref[...] = w0.astype(jnp.bfloat16)
        acc_ref[...] = jnp.zeros((_TB, 1), jnp.float32)
        iotaf_ref[...] = jax.lax.broadcasted_iota(
            jnp.int32, (_TB, _K), 1).astype(jnp.float32)

    x = lat_ref[0]
    xt = x.T

    x2 = jnp.sum(xt * xt, axis=1, keepdims=True)
    s = jax.lax.dot_general(
        xt, w_ref[...], (((1,), (1,)), ((), ())),
        preferred_element_type=jnp.float32,
        precision=jax.lax.Precision.DEFAULT)
    dist = (x2 + w2_ref[...]) - 2.0 * s

    m = jnp.min(dist, axis=1, keepdims=True)
    cand = jnp.where(dist == m, iotaf_ref[...], jnp.float32(_K))
    idx_f = jnp.min(cand, axis=1, keepdims=True)
    idx_ref[0, 0, :] = idx_f[:, 0].astype(jnp.int32)

    onehot = (cand == idx_f).astype(jnp.bfloat16)
    q = jax.lax.dot_general(
        wbf_ref[...], onehot, (((0,), (1,)), ((), ())),
        preferred_element_type=jnp.float32)
    q_ref[0] = q

    acc_ref[...] += m


def kernel(latents, weight):
    B, D, T = latents.shape
    nt = T // _TB
    q, idx3, acc = pl.pallas_call(
        _vq_block,
        grid=(B, nt),
        in_specs=[
            pl.BlockSpec((1, D, _TB), lambda b, t: (b, 0, t)),
            pl.BlockSpec((_K, _D), lambda b, t: (0, 0)),
        ],
        out_specs=[
            pl.BlockSpec((1, D, _TB), lambda b, t: (b, 0, t)),
            pl.BlockSpec((1, 1, _TB), lambda b, t: (b, 0, t)),
            pl.BlockSpec((_TB, 1), lambda b, t: (0, 0)),
        ],
        out_shape=[
            jax.ShapeDtypeStruct((B, D, T), jnp.float32),
            jax.ShapeDtypeStruct((B, 1, T), jnp.int32),
            jax.ShapeDtypeStruct((_TB, 1), jnp.float32),
        ],
        scratch_shapes=[
            pltpu.VMEM((1, _K), jnp.float32),
            pltpu.VMEM((_K, _D), jnp.bfloat16),
            pltpu.VMEM((_TB, _K), jnp.float32),
        ],
    )(latents, weight)
    mean_sq = jnp.sum(acc) / (B * T * D)
    loss = mean_sq + _BETA * mean_sq
    return q, loss, idx3.reshape(B, T)


# speedup vs baseline: 1.3117x; 1.3117x over previous
import jax
import jax.numpy as jnp
from jax.experimental import pallas as pl
from jax.experimental.pallas import tpu as pltpu

_K = 1024
_D = 64
_BETA = 0.25
_TB = 2048


def _vq_block(lat_ref, w_ref, q_ref, idx_ref, acc_ref, w2_ref, wbf_ref):
    b = pl.program_id(0)
    t = pl.program_id(1)

    @pl.when(jnp.logical_and(b == 0, t == 0))
    def _():
        w0 = w_ref[...]
        w2_ref[...] = jnp.sum(w0 * w0, axis=1)[:, None]
        wbf_ref[...] = w0.astype(jnp.bfloat16)
        acc_ref[...] = jnp.zeros((1, _TB), jnp.float32)

    x = lat_ref[0]

    x2 = jnp.sum(x * x, axis=0, keepdims=True)
    s = jax.lax.dot_general(
        w_ref[...], x, (((1,), (0,)), ((), ())),
        preferred_element_type=jnp.float32,
        precision=jax.lax.Precision.DEFAULT)
    dist = (x2 + w2_ref[...]) - 2.0 * s

    m = jnp.min(dist, axis=0, keepdims=True)
    iota = jax.lax.broadcasted_iota(jnp.int32, dist.shape, 0)
    idx = jnp.min(jnp.where(dist == m, iota, _K), axis=0)
    idx_ref[0, 0, :] = idx

    onehot = (iota == idx[None, :]).astype(jnp.bfloat16)
    q = jax.lax.dot_general(
        wbf_ref[...], onehot, (((0,), (0,)), ((), ())),
        preferred_element_type=jnp.float32)
    q_ref[0] = q

    acc_ref[...] += m


def kernel(latents, weight):
    B, D, T = latents.shape
    nt = T // _TB
    q, idx3, acc = pl.pallas_call(
        _vq_block,
        grid=(B, nt),
        in_specs=[
            pl.BlockSpec((1, D, _TB), lambda b, t: (b, 0, t)),
            pl.BlockSpec((_K, _D), lambda b, t: (0, 0)),
        ],
        out_specs=[
            pl.BlockSpec((1, D, _TB), lambda b, t: (b, 0, t)),
            pl.BlockSpec((1, 1, _TB), lambda b, t: (b, 0, t)),
            pl.BlockSpec((1, _TB), lambda b, t: (0, 0)),
        ],
        out_shape=[
            jax.ShapeDtypeStruct((B, D, T), jnp.float32),
            jax.ShapeDtypeStruct((B, 1, T), jnp.int32),
            jax.ShapeDtypeStruct((1, _TB), jnp.float32),
        ],
        scratch_shapes=[
            pltpu.VMEM((_K, 1), jnp.float32),
            pltpu.VMEM((_K, _D), jnp.bfloat16),
        ],
    )(latents, weight)
    mean_sq = jnp.sum(acc) / (B * T * D)
    loss = mean_sq + _BETA * mean_sq
    return q, loss, idx3.reshape(B, T)


# speedup vs baseline: 1.3349x; 1.0177x over previous
import jax
import jax.numpy as jnp
from jax.experimental import pallas as pl
from jax.experimental.pallas import tpu as pltpu

_K = 1024
_D = 64
_BETA = 0.25
_TB = 2048


def _vq_block(lat_ref, w_ref, q_ref, idx_ref, acc_ref, w2_ref, wbf_ref,
              wneg2_ref, iotaf_ref):
    b = pl.program_id(0)
    t = pl.program_id(1)

    @pl.when(jnp.logical_and(b == 0, t == 0))
    def _():
        w0 = w_ref[...]
        w2_ref[...] = jnp.sum(w0 * w0, axis=1)[:, None]
        wbf_ref[...] = w0.astype(jnp.bfloat16)
        wneg2_ref[...] = w0 * -2.0
        acc_ref[...] = jnp.zeros((1, _TB), jnp.float32)
        iotaf_ref[...] = jax.lax.broadcasted_iota(
            jnp.int32, (_K, _TB), 0).astype(jnp.float32)

    x = lat_ref[0]

    x2 = jnp.sum(x * x, axis=0, keepdims=True)
    sn2 = jax.lax.dot_general(
        wneg2_ref[...], x, (((1,), (0,)), ((), ())),
        preferred_element_type=jnp.float32,
        precision=jax.lax.Precision.DEFAULT)
    dist = (x2 + w2_ref[...]) + sn2

    m = jnp.min(dist, axis=0, keepdims=True)
    cand = jnp.where(dist == m, iotaf_ref[...], jnp.float32(_K))
    idx_f = jnp.min(cand, axis=0, keepdims=True)
    idx_ref[0, 0, :] = idx_f[0, :].astype(jnp.int32)

    onehot = (cand == idx_f).astype(jnp.bfloat16)
    q = jax.lax.dot_general(
        wbf_ref[...], onehot, (((0,), (0,)), ((), ())),
        preferred_element_type=jnp.float32)
    q_ref[0] = q

    acc_ref[...] += m


def kernel(latents, weight):
    B, D, T = latents.shape
    nt = T // _TB
    q, idx3, acc = pl.pallas_call(
        _vq_block,
        grid=(B, nt),
        in_specs=[
            pl.BlockSpec((1, D, _TB), lambda b, t: (b, 0, t)),
            pl.BlockSpec((_K, _D), lambda b, t: (0, 0)),
        ],
        out_specs=[
            pl.BlockSpec((1, D, _TB), lambda b, t: (b, 0, t)),
            pl.BlockSpec((1, 1, _TB), lambda b, t: (b, 0, t)),
            pl.BlockSpec((1, _TB), lambda b, t: (0, 0)),
        ],
        out_shape=[
            jax.ShapeDtypeStruct((B, D, T), jnp.float32),
            jax.ShapeDtypeStruct((B, 1, T), jnp.int32),
            jax.ShapeDtypeStruct((1, _TB), jnp.float32),
        ],
        scratch_shapes=[
            pltpu.VMEM((_K, 1), jnp.float32),
            pltpu.VMEM((_K, _D), jnp.bfloat16),
            pltpu.VMEM((_K, _D), jnp.float32),
            pltpu.VMEM((_K, _TB), jnp.float32),
        ],
    )(latents, weight)
    mean_sq = jnp.sum(acc) / (B * T * D)
    loss = mean_sq + _BETA * mean_sq
    return q, loss, idx3.reshape(B, T)


# speedup vs baseline: 1.4205x; 1.0641x over previous
import jax
import jax.numpy as jnp
from jax.experimental import pallas as pl
from jax.experimental.pallas import tpu as pltpu

_K = 1024
_D = 64
_BETA = 0.25
_TB = 2048


def _vq_block(lat_ref, w_ref, q_ref, idx_ref, acc_ref, w2_ref, wbf_ref,
              wneg2_ref, iotaf_ref):
    b = pl.program_id(0)
    t = pl.program_id(1)

    @pl.when(jnp.logical_and(b == 0, t == 0))
    def _():
        w0 = w_ref[...]
        w2_ref[...] = jnp.sum(w0 * w0, axis=1)[:, None]
        wbf_ref[...] = w0.astype(jnp.bfloat16)
        wneg2_ref[...] = w0 * -2.0
        acc_ref[...] = jnp.zeros((1, _TB), jnp.float32)
        iotaf_ref[...] = jax.lax.broadcasted_iota(
            jnp.int32, (_K, _TB), 0).astype(jnp.float32)

    x = lat_ref[0]

    x2 = jnp.sum(x * x, axis=0, keepdims=True)
    sn2 = jax.lax.dot_general(
        wneg2_ref[...], x, (((1,), (0,)), ((), ())),
        preferred_element_type=jnp.float32,
        precision=jax.lax.Precision.DEFAULT)
    dist = (x2 + w2_ref[...]) + sn2

    m = jnp.min(dist, axis=0, keepdims=True)
    cand = jnp.where(dist == m, iotaf_ref[...], jnp.float32(_K))
    idx_f = jnp.min(cand, axis=0, keepdims=True)
    idx = idx_f.astype(jnp.int32)
    idx_ref[0, 0, :] = idx[0, :]

    iota = jax.lax.broadcasted_iota(jnp.int32, (_K, _TB), 0)
    onehot = (iota == idx).astype(jnp.bfloat16)
    q = jax.lax.dot_general(
        wbf_ref[...], onehot, (((0,), (0,)), ((), ())),
        preferred_element_type=jnp.float32)
    q_ref[0] = q

    acc_ref[...] += m


def kernel(latents, weight):
    B, D, T = latents.shape
    nt = T // _TB
    q, idx3, acc = pl.pallas_call(
        _vq_block,
        grid=(B, nt),
        in_specs=[
            pl.BlockSpec((1, D, _TB), lambda b, t: (b, 0, t)),
            pl.BlockSpec((_K, _D), lambda b, t: (0, 0)),
        ],
        out_specs=[
            pl.BlockSpec((1, D, _TB), lambda b, t: (b, 0, t)),
            pl.BlockSpec((1, 1, _TB), lambda b, t: (b, 0, t)),
            pl.BlockSpec((1, _TB), lambda b, t: (0, 0)),
        ],
        out_shape=[
            jax.ShapeDtypeStruct((B, D, T), jnp.float32),
            jax.ShapeDtypeStruct((B, 1, T), jnp.int32),
            jax.ShapeDtypeStruct((1, _TB), jnp.float32),
        ],
        scratch_shapes=[
            pltpu.VMEM((_K, 1), jnp.float32),
            pltpu.VMEM((_K, _D), jnp.bfloat16),
            pltpu.VMEM((_K, _D), jnp.float32),
            pltpu.VMEM((_K, _TB), jnp.float32),
        ],
    )(latents, weight)
    mean_sq = jnp.sum(acc) / (B * T * D)
    loss = mean_sq + _BETA * mean_sq
    return q, loss, idx3.reshape(B, T)
